# Initial kernel scaffold; baseline (speedup 1.0000x reference)
import jax
import jax.numpy as jnp
from jax.experimental import pallas as pl


def kernel(hidden_states, residual, op_norm_w, ffn_norm_w, conv_in_W, conv_W, conv_out_W, gate_W, e_bias, W1, W2):
    raise NotImplementedError("write your pallas kernel here")


# speedup vs baseline: 2.1635x; 2.1635x over previous
import functools

import jax
import jax.numpy as jnp
from jax.experimental import pallas as pl
from jax.experimental.pallas import tpu as pltpu

EPS = 1e-05
TOPK = 2

S = 2048
D = 1024
E = 8
FF = 1024
ROWS_A = 256
ROWS_M = 512


def _rms(x, w):
    var = jnp.mean(x * x, axis=-1, keepdims=True)
    return x * jax.lax.rsqrt(var + EPS) * w


def _dot_t(a, b):
    return jax.lax.dot_general(a, b, (((1,), (1,)), ((), ())),
                               preferred_element_type=jnp.float32)


def _pre_moe_body(hs_ref, res_ref, hs_prev_ref, res_prev_ref, opw_ref,
                  ffw_ref, cin_ref, convt_ref, cout_ref, gw_ref, eb_ref,
                  res_out_ref, h2_ref, scores_ref, choice_ref):
    i = pl.program_id(0)
    z = hs_ref[...] + res_ref[...]
    h = _rms(z, opw_ref[...])
    bcx = _dot_t(h, cin_ref[...])
    b = bcx[:, :D]
    c = bcx[:, D:2 * D]
    x = bcx[:, 2 * D:]
    bx = b * x
    zp = hs_prev_ref[ROWS_A - 2:, :] + res_prev_ref[ROWS_A - 2:, :]
    hp = _rms(zp, opw_ref[...])
    bxh = _dot_t(hp, cin_ref[:D, :]) * _dot_t(hp, cin_ref[2 * D:, :])
    bxh = jnp.where(i > 0, bxh, 0.0)
    bxp = jnp.concatenate([bxh, bx], axis=0)
    conv = (bxp[0:ROWS_A] * convt_ref[0:1, :]
            + bxp[1:ROWS_A + 1] * convt_ref[1:2, :]
            + bxp[2:ROWS_A + 2] * convt_ref[2:3, :])
    y = c * conv
    z2 = _dot_t(y, cout_ref[...]) + z
    res_out_ref[...] = z2
    h2 = _rms(z2, ffw_ref[...])
    h2_ref[...] = h2
    logits_t = jax.lax.dot_general(gw_ref[...], h2, (((1,), (1,)), ((), ())),
                                   preferred_element_type=jnp.float32)
    sc = jax.nn.sigmoid(logits_t)
    scores_ref[...] = sc
    choice_ref[...] = sc + eb_ref[...]


def _pre_moe(hs, res, opw, ffw, cin, convt, cout, gw, eb):
    n = S // ROWS_A
    grid = (n,)
    row_blk = pl.BlockSpec((ROWS_A, D), lambda i: (i, 0))
    prev_blk = pl.BlockSpec((ROWS_A, D), lambda i: (jnp.maximum(i - 1, 0), 0))
    const2 = lambda shape: pl.BlockSpec(shape, lambda i: (0, 0))
    out_shapes = [
        jax.ShapeDtypeStruct((S, D), jnp.float32),
        jax.ShapeDtypeStruct((S, D), jnp.float32),
        jax.ShapeDtypeStruct((E, S), jnp.float32),
        jax.ShapeDtypeStruct((E, S), jnp.float32),
    ]
    return pl.pallas_call(
        _pre_moe_body,
        grid=grid,
        in_specs=[row_blk, row_blk, prev_blk, prev_blk,
                  const2((1, D)), const2((1, D)), const2((3 * D, D)),
                  const2((3, D)), const2((D, D)), const2((E, D)),
                  const2((E, 1))],
        out_specs=[row_blk, row_blk,
                   pl.BlockSpec((E, ROWS_A), lambda i: (0, i)),
                   pl.BlockSpec((E, ROWS_A), lambda i: (0, i))],
        out_shape=out_shapes,
    )(hs, res, hs, res, opw.reshape(1, D), ffw.reshape(1, D), cin, convt,
      cout, gw, eb.reshape(E, 1))


def _top2_combine(choice_t, scores_t, e):
    m1 = choice_t[0:1, :]
    s1 = scores_t[0:1, :]
    i1 = jnp.zeros_like(m1, dtype=jnp.int32)
    for k in range(1, E):
        ck = choice_t[k:k + 1, :]
        upd = ck > m1
        i1 = jnp.where(upd, k, i1)
        s1 = jnp.where(upd, scores_t[k:k + 1, :], s1)
        m1 = jnp.where(upd, ck, m1)
    m2 = jnp.full_like(m1, -jnp.inf)
    s2 = jnp.zeros_like(m1)
    i2 = jnp.zeros_like(i1)
    for k in range(E):
        ck = choice_t[k:k + 1, :]
        upd = jnp.logical_and(i1 != k, ck > m2)
        i2 = jnp.where(upd, k, i2)
        s2 = jnp.where(upd, scores_t[k:k + 1, :], s2)
        m2 = jnp.where(upd, ck, m2)
    denom = s1 + s2
    w1 = s1 / denom
    w2 = s2 / denom
    return (jnp.where(i1 == e, w1, 0.0) + jnp.where(i2 == e, w2, 0.0))


def _dense_moe_body(h2_ref, scores_ref, choice_ref, w1_ref, w2_ref,
                    out_ref, acc_ref):
    e = pl.program_id(1)
    flat = h2_ref[...]
    gu = _dot_t(flat, w1_ref[0])
    g = gu[:, :FF]
    u = gu[:, FF:]
    a = g * jax.nn.sigmoid(g) * u
    part = _dot_t(a, w2_ref[0])
    comb = _top2_combine(choice_ref[...], scores_ref[...], e)
    cur = part * comb.reshape(ROWS_M, 1)

    @pl.when(e == 0)
    def _():
        acc_ref[...] = cur

    @pl.when(e > 0)
    def _():
        acc_ref[...] += cur

    @pl.when(e == E - 1)
    def _():
        out_ref[...] = acc_ref[...]


def _dense_moe(h2, scores_t, choice_t, W1, W2):
    nt = S // ROWS_M
    return pl.pallas_call(
        _dense_moe_body,
        grid=(nt, E),
        in_specs=[
            pl.BlockSpec((ROWS_M, D), lambda t, e: (t, 0)),
            pl.BlockSpec((E, ROWS_M), lambda t, e: (0, t)),
            pl.BlockSpec((E, ROWS_M), lambda t, e: (0, t)),
            pl.BlockSpec((1, 2 * FF, D), lambda t, e: (e, 0, 0)),
            pl.BlockSpec((1, D, FF), lambda t, e: (e, 0, 0)),
        ],
        out_specs=pl.BlockSpec((ROWS_M, D), lambda t, e: (t, 0)),
        out_shape=jax.ShapeDtypeStruct((S, D), jnp.float32),
        scratch_shapes=[pltpu.VMEM((ROWS_M, D), jnp.float32)],
    )(h2, scores_t, choice_t, W1, W2)


def kernel(hidden_states, residual, op_norm_w, ffn_norm_w, conv_in_W,
           conv_W, conv_out_W, gate_W, e_bias, W1, W2):
    B = hidden_states.shape[0]
    hs = hidden_states.reshape(S, D)
    res = residual.reshape(S, D)
    convt = conv_W.T
    res_out, h2, scores_t, choice_t = _pre_moe(
        hs, res, op_norm_w, ffn_norm_w, conv_in_W, convt, conv_out_W,
        gate_W, e_bias)
    out = _dense_moe(h2, scores_t, choice_t, W1, W2)
    return out.reshape(B, S, D), res_out.reshape(B, S, D)
